# Initial kernel scaffold; baseline (speedup 1.0000x reference)
import jax
import jax.numpy as jnp
from jax.experimental import pallas as pl


def kernel(source, target, features, hood_coords, w, mu):
    raise NotImplementedError("write your pallas kernel here")


# speedup vs baseline: 4.3731x; 4.3731x over previous
import functools

import jax
import jax.numpy as jnp
from jax import lax
from jax.experimental import pallas as pl
from jax.experimental.pallas import tpu as pltpu
from jax.experimental.pallas import tpu_sc as plsc

N_NODES = 10000
N_EDGES = 160000
CH = 128
KP = 15
KPAD = 16

NC = 2
NS = 16
NW = NC * NS

EDGE_CHUNK = 128
E_PAD = 163840
EDGES_PER_W = E_PAD // NW
N_CHUNKS = EDGES_PER_W // EDGE_CHUNK

ACC_ROWS = 10240
ROWS_PER_W = ACC_ROWS // NS
PAD_TARGET = N_NODES

GIDX_BR = 160
H_BN = 400


def _h_matmul_body(f_ref, w2_ref, o_ref):
    o_ref[...] = jnp.dot(f_ref[...], w2_ref[...],
                         preferred_element_type=jnp.float32)


def _gidx_body(h_ref, s_ref, mu_ref, o_ref):
    hx = h_ref[0]
    hy = h_ref[1]
    hz = h_ref[2]
    best = jnp.full(hx.shape, jnp.inf, jnp.float32)
    bidx = jnp.zeros(hx.shape, jnp.int32)
    for k in range(KP):
        dx = hx - mu_ref[k, 0]
        dy = hy - mu_ref[k, 1]
        dz = hz - mu_ref[k, 2]
        d = jnp.sqrt(dx * dx + dy * dy + dz * dz)
        m = d < best
        best = jnp.where(m, d, best)
        bidx = jnp.where(m, k, bidx)
    o_ref[...] = s_ref[...] * KPAD + bidx


def _sc_gather_scatter(h_flat, gidx, tgt):
    mesh = plsc.VectorSubcoreMesh(core_axis_name="c", subcore_axis_name="s")

    @functools.partial(
        pl.kernel,
        out_type=jax.ShapeDtypeStruct((NC * ACC_ROWS, CH), jnp.float32),
        mesh=mesh,
        scratch_types=[
            pltpu.VMEM((EDGE_CHUNK,), jnp.int32),
            pltpu.VMEM((EDGE_CHUNK,), jnp.int32),
            pltpu.VMEM((EDGE_CHUNK, CH), jnp.float32),
            pltpu.VMEM_SHARED((ACC_ROWS, CH), jnp.float32),
            pltpu.SemaphoreType.DMA,
        ],
    )
    def sc_kernel(h_hbm, gidx_hbm, tgt_hbm, out_hbm, idx_v, tgt_v, rows_v,
                  acc, sem):
        cid = lax.axis_index("c")
        sid = lax.axis_index("s")
        wid = cid * NS + sid

        @pl.loop(0, EDGE_CHUNK)
        def _(i):
            for j in range(CH // 16):
                rows_v[i, pl.ds(j * 16, 16)] = jnp.zeros((16,), jnp.float32)

        @pl.loop(0, ROWS_PER_W // EDGE_CHUNK)
        def _(t):
            pltpu.sync_copy(
                rows_v,
                acc.at[pl.ds(sid * ROWS_PER_W + t * EDGE_CHUNK, EDGE_CHUNK)])

        plsc.subcore_barrier()

        @pl.loop(0, N_CHUNKS)
        def _(c):
            base = wid * EDGES_PER_W + c * EDGE_CHUNK
            pltpu.sync_copy(gidx_hbm.at[pl.ds(base, EDGE_CHUNK)], idx_v)
            pltpu.sync_copy(tgt_hbm.at[pl.ds(base, EDGE_CHUNK)], tgt_v)
            pltpu.async_copy(h_hbm.at[idx_v], rows_v, sem).wait()
            pltpu.sync_copy(rows_v, acc.at[tgt_v], add=True)

        plsc.subcore_barrier()
        pltpu.sync_copy(
            acc.at[pl.ds(sid * ROWS_PER_W, ROWS_PER_W)],
            out_hbm.at[pl.ds(cid * ACC_ROWS + sid * ROWS_PER_W, ROWS_PER_W)])

    return sc_kernel(h_flat, gidx, tgt)


def kernel(source, target, features, hood_coords, w, mu):
    n = features.shape[0]

    w2 = w.transpose(2, 1, 0).reshape(CH, KP * CH)
    w2 = jnp.pad(w2, ((0, 0), (0, (KPAD - KP) * CH)))
    h = pl.pallas_call(
        _h_matmul_body,
        grid=(N_NODES // H_BN,),
        in_specs=[
            pl.BlockSpec((H_BN, CH), lambda i: (i, 0)),
            pl.BlockSpec((CH, KPAD * CH), lambda i: (0, 0)),
        ],
        out_specs=pl.BlockSpec((H_BN, KPAD * CH), lambda i: (i, 0)),
        out_shape=jax.ShapeDtypeStruct((N_NODES, KPAD * CH), jnp.float32),
    )(features, w2)
    h_flat = h.reshape(N_NODES * KPAD, CH)

    hood_p = jnp.pad(hood_coords, ((0, E_PAD - N_EDGES), (0, 0)))
    src_p = jnp.pad(source, (0, E_PAD - N_EDGES))
    h3 = hood_p.T.reshape(3, E_PAD // CH, CH)
    src2 = src_p.reshape(E_PAD // CH, CH)
    gidx2 = pl.pallas_call(
        _gidx_body,
        grid=(E_PAD // CH // GIDX_BR,),
        in_specs=[
            pl.BlockSpec((3, GIDX_BR, CH), lambda i: (0, i, 0)),
            pl.BlockSpec((GIDX_BR, CH), lambda i: (i, 0)),
            pl.BlockSpec(memory_space=pltpu.SMEM),
        ],
        out_specs=pl.BlockSpec((GIDX_BR, CH), lambda i: (i, 0)),
        out_shape=jax.ShapeDtypeStruct((E_PAD // CH, CH), jnp.int32),
    )(h3, src2, mu[0])
    gidx = gidx2.reshape(E_PAD)

    tgt_p = jnp.pad(target, (0, E_PAD - N_EDGES),
                    constant_values=PAD_TARGET)

    partials = _sc_gather_scatter(h_flat, gidx, tgt_p)

    return partials[:n] + partials[ACC_ROWS:ACC_ROWS + n]


# speedup vs baseline: 4.9797x; 1.1387x over previous
import functools

import jax
import jax.numpy as jnp
from jax import lax
from jax.experimental import pallas as pl
from jax.experimental.pallas import tpu as pltpu
from jax.experimental.pallas import tpu_sc as plsc

N_NODES = 10000
N_EDGES = 160000
CH = 128
KP = 15
KPAD = 16

NC = 2
NS = 16
NW = NC * NS

EDGE_CHUNK = 128
E_PAD = 163840
EDGES_PER_W = E_PAD // NW
N_CHUNKS = EDGES_PER_W // EDGE_CHUNK

ACC_ROWS = 10240
ROWS_PER_W = ACC_ROWS // NS
PAD_TARGET = N_NODES

GIDX_BR = 160
H_BN = 400


def _h_matmul_body(f_ref, w2_ref, o_ref):
    o_ref[...] = jnp.dot(f_ref[...], w2_ref[...],
                         preferred_element_type=jnp.float32)


def _gidx_body(h_ref, s_ref, mu_ref, o_ref):
    hx = h_ref[0]
    hy = h_ref[1]
    hz = h_ref[2]
    best = jnp.full(hx.shape, jnp.inf, jnp.float32)
    bidx = jnp.zeros(hx.shape, jnp.int32)
    for k in range(KP):
        dx = hx - mu_ref[k, 0]
        dy = hy - mu_ref[k, 1]
        dz = hz - mu_ref[k, 2]
        d = jnp.sqrt(dx * dx + dy * dy + dz * dz)
        m = d < best
        best = jnp.where(m, d, best)
        bidx = jnp.where(m, k, bidx)
    o_ref[...] = s_ref[...] * KPAD + bidx


def _sc_gather_scatter(h_flat, gidx2d, tgt2d):
    mesh = plsc.VectorSubcoreMesh(core_axis_name="c", subcore_axis_name="s")

    @functools.partial(
        pl.kernel,
        out_type=jax.ShapeDtypeStruct((NC * ACC_ROWS, CH), jnp.float32),
        mesh=mesh,
        scratch_types=[
            pltpu.VMEM((N_CHUNKS, EDGE_CHUNK), jnp.int32),
            pltpu.VMEM((N_CHUNKS, EDGE_CHUNK), jnp.int32),
            pltpu.VMEM((EDGE_CHUNK, CH), jnp.float32),
            pltpu.VMEM((EDGE_CHUNK, CH), jnp.float32),
            pltpu.VMEM_SHARED((ACC_ROWS, CH), jnp.float32),
            pltpu.SemaphoreType.DMA,
            pltpu.SemaphoreType.DMA,
        ],
    )
    def sc_kernel(h_hbm, gidx_hbm, tgt_hbm, out_hbm, idx_all, tgt_all,
                  rows_a, rows_b, acc, sem_a, sem_b):
        cid = lax.axis_index("c")
        sid = lax.axis_index("s")
        wid = cid * NS + sid

        pltpu.sync_copy(gidx_hbm.at[pl.ds(wid * N_CHUNKS, N_CHUNKS)], idx_all)
        pltpu.sync_copy(tgt_hbm.at[pl.ds(wid * N_CHUNKS, N_CHUNKS)], tgt_all)

        @pl.loop(0, EDGE_CHUNK)
        def _(i):
            for j in range(CH // 16):
                rows_a[i, pl.ds(j * 16, 16)] = jnp.zeros((16,), jnp.float32)

        @pl.loop(0, ROWS_PER_W // EDGE_CHUNK)
        def _(t):
            pltpu.sync_copy(
                rows_a,
                acc.at[pl.ds(sid * ROWS_PER_W + t * EDGE_CHUNK, EDGE_CHUNK)])

        plsc.subcore_barrier()

        pltpu.async_copy(h_hbm.at[idx_all.at[0]], rows_a, sem_a)

        @pl.loop(0, N_CHUNKS // 2)
        def _(p):
            c = 2 * p
            pltpu.async_copy(h_hbm.at[idx_all.at[c + 1]], rows_b, sem_b)
            pltpu.make_async_copy(h_hbm.at[idx_all.at[c]], rows_a, sem_a).wait()
            pltpu.sync_copy(rows_a, acc.at[tgt_all.at[c]], add=True)

            @pl.when(p < N_CHUNKS // 2 - 1)
            def _():
                pltpu.async_copy(h_hbm.at[idx_all.at[c + 2]], rows_a, sem_a)

            pltpu.make_async_copy(h_hbm.at[idx_all.at[c + 1]], rows_b,
                                  sem_b).wait()
            pltpu.sync_copy(rows_b, acc.at[tgt_all.at[c + 1]], add=True)

        plsc.subcore_barrier()
        pltpu.sync_copy(
            acc.at[pl.ds(sid * ROWS_PER_W, ROWS_PER_W)],
            out_hbm.at[pl.ds(cid * ACC_ROWS + sid * ROWS_PER_W, ROWS_PER_W)])

    return sc_kernel(h_flat, gidx2d, tgt2d)


def kernel(source, target, features, hood_coords, w, mu):
    n = features.shape[0]

    w2 = w.transpose(2, 1, 0).reshape(CH, KP * CH)
    w2 = jnp.pad(w2, ((0, 0), (0, (KPAD - KP) * CH)))
    h = pl.pallas_call(
        _h_matmul_body,
        grid=(N_NODES // H_BN,),
        in_specs=[
            pl.BlockSpec((H_BN, CH), lambda i: (i, 0)),
            pl.BlockSpec((CH, KPAD * CH), lambda i: (0, 0)),
        ],
        out_specs=pl.BlockSpec((H_BN, KPAD * CH), lambda i: (i, 0)),
        out_shape=jax.ShapeDtypeStruct((N_NODES, KPAD * CH), jnp.float32),
    )(features, w2)
    h_flat = h.reshape(N_NODES * KPAD, CH)

    hood_p = jnp.pad(hood_coords, ((0, E_PAD - N_EDGES), (0, 0)))
    src_p = jnp.pad(source, (0, E_PAD - N_EDGES))
    h3 = hood_p.T.reshape(3, E_PAD // CH, CH)
    src2 = src_p.reshape(E_PAD // CH, CH)
    gidx2 = pl.pallas_call(
        _gidx_body,
        grid=(E_PAD // CH // GIDX_BR,),
        in_specs=[
            pl.BlockSpec((3, GIDX_BR, CH), lambda i: (0, i, 0)),
            pl.BlockSpec((GIDX_BR, CH), lambda i: (i, 0)),
            pl.BlockSpec(memory_space=pltpu.SMEM),
        ],
        out_specs=pl.BlockSpec((GIDX_BR, CH), lambda i: (i, 0)),
        out_shape=jax.ShapeDtypeStruct((E_PAD // CH, CH), jnp.int32),
    )(h3, src2, mu[0])
    tgt_p = jnp.pad(target, (0, E_PAD - N_EDGES),
                    constant_values=PAD_TARGET)
    tgt2d = tgt_p.reshape(E_PAD // EDGE_CHUNK, EDGE_CHUNK)

    partials = _sc_gather_scatter(h_flat, gidx2, tgt2d)

    return partials[:n] + partials[ACC_ROWS:ACC_ROWS + n]
